# baseline (speedup 1.0000x reference)
import functools
import jax
import jax.numpy as jnp
from jax import lax
from jax.experimental import pallas as pl
from jax.experimental.pallas import tpu as pltpu, tpu_sc as plsc

N = 10000
E = 320000
D = 128
G = 128
C = 10

NC = 2
NS = 16
L = 16

ET = E // NS
COLS = N // NS
NB = 624

_mesh = plsc.VectorSubcoreMesh(core_axis_name="c", subcore_axis_name="s")
_sc_params = pltpu.CompilerParams(needs_layout_passes=False)


def _m8(x):
    return pl.multiple_of(x, 8)


@functools.partial(
    pl.kernel,
    out_type=[
        jax.ShapeDtypeStruct((NC * NS * N,), jnp.float32),
        jax.ShapeDtypeStruct((NC * NS * G,), jnp.float32),
    ],
    mesh=_mesh,
    compiler_params=_sc_params,
    scratch_types=[
        pltpu.VMEM((2000,), jnp.int32),
        pltpu.VMEM((N,), jnp.float32),
        pltpu.VMEM((G,), jnp.float32),
    ],
)
def _sc_deg(dst1, batch1, deg_out, cnt_out, stage, deg_loc, cnt_loc):
    c = lax.axis_index("c")
    s = lax.axis_index("s")
    ones = jnp.full((L,), 1.0, jnp.float32)
    zeros = jnp.zeros((L,), jnp.float32)

    def zero_deg(j, _):
        deg_loc[pl.ds(j * L, L)] = zeros
        return 0
    lax.fori_loop(0, N // L, zero_deg, 0)
    for q in range(G // L):
        cnt_loc[pl.ds(q * L, L)] = zeros

    for k in range(ET // 2000):
        pltpu.sync_copy(dst1.at[pl.ds(_m8(c * E + s * ET + k * 2000), 2000)],
                        stage)

        def body(j, _):
            d = stage[pl.ds(j * L, L)]
            plsc.addupdate_scatter(deg_loc, [d], ones)
            return 0
        lax.fori_loop(0, 2000 // L, body, 0)

    pltpu.sync_copy(batch1.at[pl.ds(_m8(c * N + s * NB), NB)],
                    stage.at[pl.ds(0, NB)])

    def bbody(j, _):
        b = stage[pl.ds(j * L, L)]
        plsc.addupdate_scatter(cnt_loc, [b], ones)
        return 0
    lax.fori_loop(0, NB // L, bbody, 0)

    @pl.when(s == NS - 1)
    def _():
        pltpu.sync_copy(batch1.at[pl.ds(_m8(c * N + NS * NB), L)],
                        stage.at[pl.ds(0, L)])
        b = stage[pl.ds(0, L)]
        plsc.addupdate_scatter(cnt_loc, [b], ones)

    pltpu.sync_copy(deg_loc, deg_out.at[pl.ds(_m8((c * NS + s) * N), N)])
    pltpu.sync_copy(cnt_loc, cnt_out.at[pl.ds(_m8((c * NS + s) * G), G)])


RPC = 125
STG = 8
OCH = COLS // RPC


NCH = ET // RPC
NSTG = NCH // STG


@functools.partial(
    pl.kernel,
    out_type=jax.ShapeDtypeStruct((NC, NS * OCH, RPC, D), jnp.float32),
    mesh=_mesh,
    compiler_params=_sc_params,
    scratch_types=[
        pltpu.VMEM((3 * STG, RPC), jnp.int32),
        pltpu.VMEM((3 * STG, RPC), jnp.int32),
        pltpu.VMEM((RPC, D), jnp.float32),
        pltpu.VMEM((RPC, D), jnp.float32),
        pltpu.VMEM_SHARED((N, D), jnp.float32),
        pltpu.SemaphoreType.DMA,
        pltpu.SemaphoreType.DMA,
        pltpu.SemaphoreType.DMA,
        pltpu.SemaphoreType.DMA,
        pltpu.SemaphoreType.DMA,
        pltpu.SemaphoreType.DMA,
        pltpu.SemaphoreType.DMA,
    ],
)
def _sc_agg(xsf, src3, dst3, zrows, agg_out, st_src, st_dst, rows0, rows1,
            acc, gs0, gs1, ss0, ss1, is0, is1, is2):
    c = lax.axis_index("c")
    s = lax.axis_index("s")

    pltpu.sync_copy(zrows, rows0)
    for i in range(OCH):
        pltpu.sync_copy(rows0, acc.at[pl.ds(s * COLS + i * RPC, RPC)])
    plsc.subcore_barrier()

    rowbufs = (rows0, rows1)
    gsems = (gs0, gs1)
    ssems = (ss0, ss1)
    isems = (is0, is1, is2)

    def stage_idx(kk):
        p = kk % 3
        sb = _m8(s * NCH + kk * STG)
        d0 = pltpu.async_copy(src3.at[c, pl.ds(sb, STG)],
                              st_src.at[pl.ds(p * STG, STG)], isems[p])
        d1 = pltpu.async_copy(dst3.at[c, pl.ds(sb, STG)],
                              st_dst.at[pl.ds(p * STG, STG)], isems[p])
        return (d0, d1)

    idx_desc = {0: stage_idx(0), 1: stage_idx(1)}
    for dsc in idx_desc[0]:
        dsc.wait()
    gat = [None, None]
    sca = [None, None]
    gat[0] = pltpu.async_copy(xsf.at[st_src.at[0]], rows0, gs0)
    for t in range(NCH):
        b = t % 2
        o = 1 - b
        gat[b].wait()
        nt = t + 1
        if nt < NCH and nt % STG == 0:
            kk = nt // STG
            for dsc in idx_desc[kk % 3]:
                dsc.wait()
            if kk + 1 < NSTG:
                idx_desc[(kk + 1) % 3] = stage_idx(kk + 1)
        if sca[o] is not None:
            sca[o].wait()
        if nt < NCH:
            row = (nt // STG) % 3 * STG + nt % STG
            gat[o] = pltpu.async_copy(xsf.at[st_src.at[row]], rowbufs[o],
                                      gsems[o])
        row = (t // STG) % 3 * STG + t % STG
        sca[b] = pltpu.async_copy(rowbufs[b], acc.at[st_dst.at[row]],
                                  ssems[b], add=True)
    sca[(NCH - 1) % 2].wait()

    plsc.subcore_barrier()
    for i in range(OCH):
        pltpu.sync_copy(acc.at[pl.ds(s * COLS + i * RPC, RPC)], rows0)
        pltpu.sync_copy(rows0, agg_out.at[c, s * OCH + i])


QCH = 2000
NCHQ = ET // QCH
NG = N * G
DUM = 0


@functools.partial(
    pl.kernel,
    out_type=jax.ShapeDtypeStruct((NC * N * G,), jnp.float32),
    mesh=_mesh,
    compiler_params=_sc_params,
    scratch_types=[
        pltpu.VMEM((QCH,), jnp.int32),
        pltpu.VMEM((QCH,), jnp.int32),
        pltpu.VMEM((16, 128), jnp.int32),
        pltpu.VMEM((16, 128), jnp.float32),
        pltpu.VMEM((16000,), jnp.float32),
        pltpu.VMEM((N,), jnp.int32),
        pltpu.VMEM((N,), jnp.float32),
        pltpu.VMEM_SHARED((NG,), jnp.float32),
        pltpu.SemaphoreType.DMA,
        pltpu.SemaphoreType.DMA,
    ],
)
def _sc_qbuild(src1, dst1, batch1, dinv1, q_out, st_s, st_d,
               qi, qv, zb, batch_loc, dinv_loc, qacc, semA, sq):
    c = lax.axis_index("c")
    s = lax.axis_index("s")
    zeros = jnp.zeros((L,), jnp.float32)
    dums = jnp.full((L,), DUM, jnp.int32)

    def zero_zb(j, _):
        zb[pl.ds(j * L, L)] = zeros
        return 0
    lax.fori_loop(0, 16000 // L, zero_zb, 0)
    for i in range(5):
        pltpu.sync_copy(zb, qacc.at[pl.ds(s * 80000 + i * 16000, 16000)])

    pltpu.sync_copy(batch1.at[pl.ds(_m8(c * N), N)], batch_loc)
    pltpu.sync_copy(dinv1.at[pl.ds(_m8(c * N), N)], dinv_loc)
    plsc.subcore_barrier()

    ebase = c * E + s * ET

    def chunk_body(k, _):
        pltpu.sync_copy(src1.at[pl.ds(_m8(ebase + k * QCH), QCH)], st_s)
        pltpu.sync_copy(dst1.at[pl.ds(_m8(ebase + k * QCH), QCH)], st_d)

        @pl.when(k > 0)
        def _():
            for r in range(16):
                pltpu.make_async_copy(qv.at[r], qacc.at[qi.at[r]], sq).wait()

        for j in range(QCH // L):
            r, col = j >> 3, (j & 7) * L
            sv = st_s[pl.ds(j * L, L)]
            dv = st_d[pl.ds(j * L, L)]
            g = plsc.load_gather(batch_loc, [dv])
            f2 = plsc.load_gather(dinv_loc, [dv])
            qi[r, pl.ds(col, L)] = lax.shift_left(sv, 7) + g
            qv[r, pl.ds(col, L)] = f2
        for j in range(QCH // L, 128):
            r, col = j >> 3, (j & 7) * L
            qi[r, pl.ds(col, L)] = dums
            qv[r, pl.ds(col, L)] = zeros

        for r in range(16):
            pltpu.async_copy(qv.at[r], qacc.at[qi.at[r]], sq, add=True)
        return 0
    lax.fori_loop(0, NCHQ, chunk_body, 0)
    for r in range(16):
        pltpu.make_async_copy(qv.at[r], qacc.at[qi.at[r]], sq).wait()

    io = lax.iota(jnp.int32, L)
    base = s * COLS
    for j in range(40):
        r, col = j >> 3, (j & 7) * L
        jv = base + j * L + io
        ok = (jv - base) < COLS
        jvc = jnp.where(jv > N - 1, N - 1, jv)
        g = plsc.load_gather(batch_loc, [jvc])
        dvv = plsc.load_gather(dinv_loc, [jvc])
        qi[r, pl.ds(col, L)] = jnp.where(ok, lax.shift_left(jv, 7) + g, DUM)
        qv[r, pl.ds(col, L)] = jnp.where(ok, dvv, 0.0)
    for r in range(5):
        pltpu.async_copy(qv.at[r], qacc.at[qi.at[r]], sq, add=True)
    for r in range(5):
        pltpu.make_async_copy(qv.at[r], qacc.at[qi.at[r]], sq).wait()

    plsc.subcore_barrier()
    for i in range(5):
        pltpu.sync_copy(qacc.at[pl.ds(s * 80000 + i * 16000, 16000)], zb)
        pltpu.sync_copy(zb, q_out.at[pl.ds(_m8(c * NG + s * 80000 + i * 16000),
                                           16000)])


BLK1 = 2000


def _tc1_body(degp_ref, x_ref, xs_ref, dinv_ref):
    deg = jnp.sum(degp_ref[0], axis=1, keepdims=True) + 1.0
    dinv = lax.rsqrt(jnp.maximum(deg, 1.0))
    xs_ref[0] = x_ref[0] * dinv
    dinv_ref[0] = dinv


def _tc_scale(deg_pT, x2):
    return pl.pallas_call(
        _tc1_body,
        grid=(NC, N // BLK1),
        in_specs=[
            pl.BlockSpec((1, BLK1, NS), lambda c, i: (c, i, 0)),
            pl.BlockSpec((1, BLK1, D), lambda c, i: (c, i, 0)),
        ],
        out_specs=[
            pl.BlockSpec((1, BLK1, D), lambda c, i: (c, i, 0)),
            pl.BlockSpec((1, BLK1, 1), lambda c, i: (c, i, 0)),
        ],
        out_shape=[
            jax.ShapeDtypeStruct((NC, N, D), jnp.float32),
            jax.ShapeDtypeStruct((NC, N, 1), jnp.float32),
        ],
    )(deg_pT, x2)


BLK2 = 2000
NSTEP = N // BLK2
EPS = 1e-7


def _bce_mean(p, y):
    p = jnp.clip(p, EPS, 1.0 - EPS)
    return -jnp.mean(y * jnp.log(p) + (1.0 - y) * jnp.log(1.0 - p))


def _tc23_body(agg_ref, xs_ref, dinv_ref, q_ref, cntT_ref, y_ref,
               w1_ref, b1_ref, w2_ref, b2_ref, wc1_ref, bc1_ref,
               wc2_ref, bc2_ref, wd_ref, bd_ref, out_ref, acc0, acc1):
    c = pl.program_id(0)
    i = pl.program_id(1)

    @pl.when((c == 0) & (i == 0))
    def _():
        acc0[...] = jnp.zeros((G, D), jnp.float32)
        acc1[...] = jnp.zeros((G, D), jnp.float32)

    a = (agg_ref[0] + xs_ref[0]) * dinv_ref[0]
    h = lax.dot_general(a, w1_ref[...], (((1,), (0,)), ((), ())),
                        preferred_element_type=jnp.float32) + b1_ref[...]
    h = jnp.maximum(h, 0.0)
    m = lax.dot_general(h, w2_ref[...], (((1,), (0,)), ((), ())),
                        preferred_element_type=jnp.float32)
    m = m * dinv_ref[0]
    qm = lax.dot_general(q_ref[0], m, (((0,), (0,)), ((), ())),
                         preferred_element_type=jnp.float32)

    @pl.when(c == 0)
    def _():
        acc0[...] += qm

    @pl.when(c == 1)
    def _():
        acc1[...] += qm

    @pl.when((c == NC - 1) & (i == NSTEP - 1))
    def _():
        cnt_s = jnp.maximum(jnp.sum(cntT_ref[0], axis=1, keepdims=True), 1.0)
        cnt_t = jnp.maximum(jnp.sum(cntT_ref[1], axis=1, keepdims=True), 1.0)
        ps = acc0[...] / cnt_s + b2_ref[...]
        pt = acc1[...] / cnt_t + b2_ref[...]

        hh = lax.dot_general(ps, wc1_ref[...], (((1,), (0,)), ((), ())),
                             preferred_element_type=jnp.float32) + bc1_ref[...]
        hh = jnp.maximum(hh, 0.0)
        z = lax.dot_general(hh, wc2_ref[...], (((1,), (0,)), ((), ())),
                            preferred_element_type=jnp.float32) + bc2_ref[...]
        logits = jax.nn.sigmoid(z)
        ycol = y_ref[...]
        onehot = (lax.broadcasted_iota(jnp.int32, (G, C), 1) == ycol
                  ).astype(jnp.float32)
        clf = _bce_mean(logits, onehot)

        sp = jax.nn.sigmoid(
            lax.dot_general(ps, wd_ref[...], (((1,), (0,)), ((), ())),
                            preferred_element_type=jnp.float32) + bd_ref[...])
        tp = jax.nn.sigmoid(
            lax.dot_general(pt, wd_ref[...], (((1,), (0,)), ((), ())),
                            preferred_element_type=jnp.float32) + bd_ref[...])
        dl = _bce_mean(sp, jnp.zeros_like(sp)) + _bce_mean(tp, jnp.ones_like(tp))
        total = clf + dl
        out_ref[...] = jnp.stack([total, clf, dl]).reshape(1, 3)


def _tc_fused(agg, xs2, dinvT, QTo, cntT, y2, W1, b1r, W2, b2r,
              Wc1, bc1r, Wc2, bc2r, Wd, bdr):
    return pl.pallas_call(
        _tc23_body,
        grid=(NC, NSTEP),
        in_specs=[
            pl.BlockSpec((1, BLK2, D), lambda c, i: (c, i, 0)),
            pl.BlockSpec((1, BLK2, D), lambda c, i: (c, i, 0)),
            pl.BlockSpec((1, BLK2, 1), lambda c, i: (c, i, 0)),
            pl.BlockSpec((1, BLK2, G), lambda c, i: (c, i, 0)),
            pl.BlockSpec((NC, G, NS), lambda c, i: (0, 0, 0)),
            pl.BlockSpec((G, 1), lambda c, i: (0, 0)),
            pl.BlockSpec((D, D), lambda c, i: (0, 0)),
            pl.BlockSpec((1, D), lambda c, i: (0, 0)),
            pl.BlockSpec((D, D), lambda c, i: (0, 0)),
            pl.BlockSpec((1, D), lambda c, i: (0, 0)),
            pl.BlockSpec((D, 16), lambda c, i: (0, 0)),
            pl.BlockSpec((1, 16), lambda c, i: (0, 0)),
            pl.BlockSpec((16, C), lambda c, i: (0, 0)),
            pl.BlockSpec((1, C), lambda c, i: (0, 0)),
            pl.BlockSpec((D, 1), lambda c, i: (0, 0)),
            pl.BlockSpec((1, 1), lambda c, i: (0, 0)),
        ],
        out_specs=pl.BlockSpec((1, 3), lambda c, i: (0, 0)),
        out_shape=jax.ShapeDtypeStruct((1, 3), jnp.float32),
        scratch_shapes=[pltpu.VMEM((G, D), jnp.float32),
                        pltpu.VMEM((G, D), jnp.float32)],
    )(agg, xs2, dinvT, QTo, cntT, y2, W1, b1r, W2, b2r,
      Wc1, bc1r, Wc2, bc2r, Wd, bdr)


def kernel(src_x, src_edge_index, src_batch, src_y, tgt_x, tgt_edge_index,
           tgt_batch, W1, b1, W2, b2, Wc1, bc1, Wc2, bc2, Wd, bd):
    srcf = jnp.stack([src_edge_index[0], tgt_edge_index[0]])
    dstf = jnp.stack([src_edge_index[1], tgt_edge_index[1]])
    src1 = srcf.reshape(-1)
    dst1 = dstf.reshape(-1)
    batch1 = jnp.concatenate([src_batch, tgt_batch])
    x2 = jnp.stack([src_x, tgt_x])

    deg_flat, cnt_flat = _sc_deg(dst1, batch1)
    deg_pT = jnp.transpose(deg_flat.reshape(NC, NS, N), (0, 2, 1))
    cntT = jnp.transpose(cnt_flat.reshape(NC, NS, G), (0, 2, 1))

    xs2, dinvT = _tc_scale(deg_pT, x2)
    dinv1 = dinvT.reshape(-1)
    xsf = xs2.reshape(NC * N, D)
    off = jnp.array([[0], [N]], jnp.int32)
    src3 = (srcf + off).reshape(NC, E // RPC, RPC)
    dst3 = dstf.reshape(NC, E // RPC, RPC)
    zrows = jnp.zeros((RPC, D), jnp.float32)

    agg = _sc_agg(xsf, src3, dst3, zrows).reshape(NC, N, D)

    QTo = _sc_qbuild(src1, dst1, batch1, dinv1).reshape(NC, N, G)

    out = _tc_fused(agg, xs2, dinvT, QTo, cntT,
                    src_y.reshape(G, 1).astype(jnp.int32),
                    W1, b1.reshape(1, D), W2, b2.reshape(1, D),
                    Wc1, bc1.reshape(1, 16), Wc2, bc2.reshape(1, C),
                    Wd, bd.reshape(1, 1))
    return (out[0, 0], out[0, 1], out[0, 2])


# speedup vs baseline: 1.0032x; 1.0032x over previous
import functools
import jax
import jax.numpy as jnp
from jax import lax
from jax.experimental import pallas as pl
from jax.experimental.pallas import tpu as pltpu, tpu_sc as plsc

N = 10000
E = 320000
D = 128
G = 128
C = 10

NC = 2
NS = 16
L = 16

ET = E // NS
COLS = N // NS
NB = 624

_mesh = plsc.VectorSubcoreMesh(core_axis_name="c", subcore_axis_name="s")
_sc_params = pltpu.CompilerParams(needs_layout_passes=False)


def _m8(x):
    return pl.multiple_of(x, 8)


@functools.partial(
    pl.kernel,
    out_type=[
        jax.ShapeDtypeStruct((NC * NS * N,), jnp.float32),
        jax.ShapeDtypeStruct((NC * NS * G,), jnp.float32),
    ],
    mesh=_mesh,
    compiler_params=_sc_params,
    scratch_types=[
        pltpu.VMEM((2000,), jnp.int32),
        pltpu.VMEM((N,), jnp.float32),
        pltpu.VMEM((G,), jnp.float32),
    ],
)
def _sc_deg(dst1, batch1, deg_out, cnt_out, stage, deg_loc, cnt_loc):
    c = lax.axis_index("c")
    s = lax.axis_index("s")
    ones = jnp.full((L,), 1.0, jnp.float32)
    zeros = jnp.zeros((L,), jnp.float32)

    def zero_deg(j, _):
        deg_loc[pl.ds(j * L, L)] = zeros
        return 0
    lax.fori_loop(0, N // L, zero_deg, 0)
    for q in range(G // L):
        cnt_loc[pl.ds(q * L, L)] = zeros

    for k in range(ET // 2000):
        pltpu.sync_copy(dst1.at[pl.ds(_m8(c * E + s * ET + k * 2000), 2000)],
                        stage)

        def body(j, _):
            d = stage[pl.ds(j * L, L)]
            plsc.addupdate_scatter(deg_loc, [d], ones)
            return 0
        lax.fori_loop(0, 2000 // L, body, 0)

    pltpu.sync_copy(batch1.at[pl.ds(_m8(c * N + s * NB), NB)],
                    stage.at[pl.ds(0, NB)])

    def bbody(j, _):
        b = stage[pl.ds(j * L, L)]
        plsc.addupdate_scatter(cnt_loc, [b], ones)
        return 0
    lax.fori_loop(0, NB // L, bbody, 0)

    @pl.when(s == NS - 1)
    def _():
        pltpu.sync_copy(batch1.at[pl.ds(_m8(c * N + NS * NB), L)],
                        stage.at[pl.ds(0, L)])
        b = stage[pl.ds(0, L)]
        plsc.addupdate_scatter(cnt_loc, [b], ones)

    pltpu.sync_copy(deg_loc, deg_out.at[pl.ds(_m8((c * NS + s) * N), N)])
    pltpu.sync_copy(cnt_loc, cnt_out.at[pl.ds(_m8((c * NS + s) * G), G)])


RPC = 125
STG = 8
OCH = COLS // RPC


NCH = ET // RPC
NSTG = NCH // STG


@functools.partial(
    pl.kernel,
    out_type=jax.ShapeDtypeStruct((NC, NS * OCH, RPC, D), jnp.float32),
    mesh=_mesh,
    compiler_params=_sc_params,
    scratch_types=[
        pltpu.VMEM((3 * STG, RPC), jnp.int32),
        pltpu.VMEM((3 * STG, RPC), jnp.int32),
        pltpu.VMEM((RPC, D), jnp.float32),
        pltpu.VMEM((RPC, D), jnp.float32),
        pltpu.VMEM_SHARED((N, D), jnp.float32),
        pltpu.SemaphoreType.DMA,
        pltpu.SemaphoreType.DMA,
        pltpu.SemaphoreType.DMA,
        pltpu.SemaphoreType.DMA,
        pltpu.SemaphoreType.DMA,
        pltpu.SemaphoreType.DMA,
        pltpu.SemaphoreType.DMA,
    ],
)
def _sc_agg(xsf, src3, dst3, zrows, agg_out, st_src, st_dst, rows0, rows1,
            acc, gs0, gs1, ss0, ss1, is0, is1, is2):
    c = lax.axis_index("c")
    s = lax.axis_index("s")

    pltpu.sync_copy(zrows, rows0)
    for i in range(OCH):
        pltpu.sync_copy(rows0, acc.at[pl.ds(s * COLS + i * RPC, RPC)])
    plsc.subcore_barrier()

    rowbufs = (rows0, rows1)
    gsems = (gs0, gs1)
    ssems = (ss0, ss1)
    isems = (is0, is1, is2)

    def stage_idx(kk):
        p = kk % 3
        sb = _m8(s * NCH + kk * STG)
        d0 = pltpu.async_copy(src3.at[c, pl.ds(sb, STG)],
                              st_src.at[pl.ds(p * STG, STG)], isems[p])
        d1 = pltpu.async_copy(dst3.at[c, pl.ds(sb, STG)],
                              st_dst.at[pl.ds(p * STG, STG)], isems[p])
        return (d0, d1)

    idx_desc = {0: stage_idx(0), 1: stage_idx(1)}
    for dsc in idx_desc[0]:
        dsc.wait()
    gat = [None, None]
    sca = [None, None]
    gat[0] = pltpu.async_copy(xsf.at[st_src.at[0]], rows0, gs0)
    for t in range(NCH):
        b = t % 2
        o = 1 - b
        gat[b].wait()
        nt = t + 1
        if nt < NCH and nt % STG == 0:
            kk = nt // STG
            for dsc in idx_desc[kk % 3]:
                dsc.wait()
            if kk + 1 < NSTG:
                idx_desc[(kk + 1) % 3] = stage_idx(kk + 1)
        if sca[o] is not None:
            sca[o].wait()
        if nt < NCH:
            row = (nt // STG) % 3 * STG + nt % STG
            gat[o] = pltpu.async_copy(xsf.at[st_src.at[row]], rowbufs[o],
                                      gsems[o])
        row = (t // STG) % 3 * STG + t % STG
        sca[b] = pltpu.async_copy(rowbufs[b], acc.at[st_dst.at[row]],
                                  ssems[b], add=True)
    sca[(NCH - 1) % 2].wait()

    plsc.subcore_barrier()
    out_desc = [None, None]
    for i in range(OCH):
        bb = (rows0, rows1)[i % 2]
        if out_desc[i % 2] is not None:
            out_desc[i % 2].wait()
        pltpu.sync_copy(acc.at[pl.ds(s * COLS + i * RPC, RPC)], bb)
        out_desc[i % 2] = pltpu.async_copy(bb, agg_out.at[c, s * OCH + i], gs0)
    out_desc[1].wait()
    out_desc[0].wait()


QCH = 2000
NCHQ = ET // QCH
NG = N * G
DUM = 0


@functools.partial(
    pl.kernel,
    out_type=jax.ShapeDtypeStruct((NC * N * G,), jnp.float32),
    mesh=_mesh,
    compiler_params=_sc_params,
    scratch_types=[
        pltpu.VMEM((QCH,), jnp.int32),
        pltpu.VMEM((QCH,), jnp.int32),
        pltpu.VMEM((16, 128), jnp.int32),
        pltpu.VMEM((16, 128), jnp.float32),
        pltpu.VMEM((16000,), jnp.float32),
        pltpu.VMEM((N,), jnp.int32),
        pltpu.VMEM((N,), jnp.float32),
        pltpu.VMEM_SHARED((NG,), jnp.float32),
        pltpu.SemaphoreType.DMA,
        pltpu.SemaphoreType.DMA,
    ],
)
def _sc_qbuild(src1, dst1, batch1, dinv1, q_out, st_s, st_d,
               qi, qv, zb, batch_loc, dinv_loc, qacc, semA, sq):
    c = lax.axis_index("c")
    s = lax.axis_index("s")
    zeros = jnp.zeros((L,), jnp.float32)
    dums = jnp.full((L,), DUM, jnp.int32)

    def zero_zb(j, _):
        zb[pl.ds(j * L, L)] = zeros
        return 0
    lax.fori_loop(0, 16000 // L, zero_zb, 0)
    for i in range(5):
        pltpu.sync_copy(zb, qacc.at[pl.ds(s * 80000 + i * 16000, 16000)])

    pltpu.sync_copy(batch1.at[pl.ds(_m8(c * N), N)], batch_loc)
    pltpu.sync_copy(dinv1.at[pl.ds(_m8(c * N), N)], dinv_loc)
    plsc.subcore_barrier()

    ebase = c * E + s * ET

    def chunk_body(k, _):
        pltpu.sync_copy(src1.at[pl.ds(_m8(ebase + k * QCH), QCH)], st_s)
        pltpu.sync_copy(dst1.at[pl.ds(_m8(ebase + k * QCH), QCH)], st_d)

        @pl.when(k > 0)
        def _():
            for r in range(16):
                pltpu.make_async_copy(qv.at[r], qacc.at[qi.at[r]], sq).wait()

        for j in range(QCH // L):
            r, col = j >> 3, (j & 7) * L
            sv = st_s[pl.ds(j * L, L)]
            dv = st_d[pl.ds(j * L, L)]
            g = plsc.load_gather(batch_loc, [dv])
            f2 = plsc.load_gather(dinv_loc, [dv])
            qi[r, pl.ds(col, L)] = lax.shift_left(sv, 7) + g
            qv[r, pl.ds(col, L)] = f2
        for j in range(QCH // L, 128):
            r, col = j >> 3, (j & 7) * L
            qi[r, pl.ds(col, L)] = dums
            qv[r, pl.ds(col, L)] = zeros

        for r in range(16):
            pltpu.async_copy(qv.at[r], qacc.at[qi.at[r]], sq, add=True)
        return 0
    lax.fori_loop(0, NCHQ, chunk_body, 0)
    for r in range(16):
        pltpu.make_async_copy(qv.at[r], qacc.at[qi.at[r]], sq).wait()

    io = lax.iota(jnp.int32, L)
    base = s * COLS
    for j in range(40):
        r, col = j >> 3, (j & 7) * L
        jv = base + j * L + io
        ok = (jv - base) < COLS
        jvc = jnp.where(jv > N - 1, N - 1, jv)
        g = plsc.load_gather(batch_loc, [jvc])
        dvv = plsc.load_gather(dinv_loc, [jvc])
        qi[r, pl.ds(col, L)] = jnp.where(ok, lax.shift_left(jv, 7) + g, DUM)
        qv[r, pl.ds(col, L)] = jnp.where(ok, dvv, 0.0)
    for r in range(5):
        pltpu.async_copy(qv.at[r], qacc.at[qi.at[r]], sq, add=True)
    for r in range(5):
        pltpu.make_async_copy(qv.at[r], qacc.at[qi.at[r]], sq).wait()

    plsc.subcore_barrier()
    for i in range(5):
        pltpu.sync_copy(qacc.at[pl.ds(s * 80000 + i * 16000, 16000)], zb)
        pltpu.sync_copy(zb, q_out.at[pl.ds(_m8(c * NG + s * 80000 + i * 16000),
                                           16000)])


BLK1 = 2000


def _tc1_body(degp_ref, x_ref, xs_ref, dinv_ref):
    deg = jnp.sum(degp_ref[0], axis=1, keepdims=True) + 1.0
    dinv = lax.rsqrt(jnp.maximum(deg, 1.0))
    xs_ref[0] = x_ref[0] * dinv
    dinv_ref[0] = dinv


def _tc_scale(deg_pT, x2):
    return pl.pallas_call(
        _tc1_body,
        grid=(NC, N // BLK1),
        in_specs=[
            pl.BlockSpec((1, BLK1, NS), lambda c, i: (c, i, 0)),
            pl.BlockSpec((1, BLK1, D), lambda c, i: (c, i, 0)),
        ],
        out_specs=[
            pl.BlockSpec((1, BLK1, D), lambda c, i: (c, i, 0)),
            pl.BlockSpec((1, BLK1, 1), lambda c, i: (c, i, 0)),
        ],
        out_shape=[
            jax.ShapeDtypeStruct((NC, N, D), jnp.float32),
            jax.ShapeDtypeStruct((NC, N, 1), jnp.float32),
        ],
    )(deg_pT, x2)


BLK2 = 2000
NSTEP = N // BLK2
EPS = 1e-7


def _bce_mean(p, y):
    p = jnp.clip(p, EPS, 1.0 - EPS)
    return -jnp.mean(y * jnp.log(p) + (1.0 - y) * jnp.log(1.0 - p))


def _tc23_body(agg_ref, xs_ref, dinv_ref, q_ref, cntT_ref, y_ref,
               w1_ref, b1_ref, w2_ref, b2_ref, wc1_ref, bc1_ref,
               wc2_ref, bc2_ref, wd_ref, bd_ref, out_ref, acc0, acc1):
    c = pl.program_id(0)
    i = pl.program_id(1)

    @pl.when((c == 0) & (i == 0))
    def _():
        acc0[...] = jnp.zeros((G, D), jnp.float32)
        acc1[...] = jnp.zeros((G, D), jnp.float32)

    a = (agg_ref[0] + xs_ref[0]) * dinv_ref[0]
    h = lax.dot_general(a, w1_ref[...], (((1,), (0,)), ((), ())),
                        preferred_element_type=jnp.float32) + b1_ref[...]
    h = jnp.maximum(h, 0.0)
    m = lax.dot_general(h, w2_ref[...], (((1,), (0,)), ((), ())),
                        preferred_element_type=jnp.float32)
    m = m * dinv_ref[0]
    qm = lax.dot_general(q_ref[0], m, (((0,), (0,)), ((), ())),
                         preferred_element_type=jnp.float32)

    @pl.when(c == 0)
    def _():
        acc0[...] += qm

    @pl.when(c == 1)
    def _():
        acc1[...] += qm

    @pl.when((c == NC - 1) & (i == NSTEP - 1))
    def _():
        cnt_s = jnp.maximum(jnp.sum(cntT_ref[0], axis=1, keepdims=True), 1.0)
        cnt_t = jnp.maximum(jnp.sum(cntT_ref[1], axis=1, keepdims=True), 1.0)
        ps = acc0[...] / cnt_s + b2_ref[...]
        pt = acc1[...] / cnt_t + b2_ref[...]

        hh = lax.dot_general(ps, wc1_ref[...], (((1,), (0,)), ((), ())),
                             preferred_element_type=jnp.float32) + bc1_ref[...]
        hh = jnp.maximum(hh, 0.0)
        z = lax.dot_general(hh, wc2_ref[...], (((1,), (0,)), ((), ())),
                            preferred_element_type=jnp.float32) + bc2_ref[...]
        logits = jax.nn.sigmoid(z)
        ycol = y_ref[...]
        onehot = (lax.broadcasted_iota(jnp.int32, (G, C), 1) == ycol
                  ).astype(jnp.float32)
        clf = _bce_mean(logits, onehot)

        sp = jax.nn.sigmoid(
            lax.dot_general(ps, wd_ref[...], (((1,), (0,)), ((), ())),
                            preferred_element_type=jnp.float32) + bd_ref[...])
        tp = jax.nn.sigmoid(
            lax.dot_general(pt, wd_ref[...], (((1,), (0,)), ((), ())),
                            preferred_element_type=jnp.float32) + bd_ref[...])
        dl = _bce_mean(sp, jnp.zeros_like(sp)) + _bce_mean(tp, jnp.ones_like(tp))
        total = clf + dl
        out_ref[...] = jnp.stack([total, clf, dl]).reshape(1, 3)


def _tc_fused(agg, xs2, dinvT, QTo, cntT, y2, W1, b1r, W2, b2r,
              Wc1, bc1r, Wc2, bc2r, Wd, bdr):
    return pl.pallas_call(
        _tc23_body,
        grid=(NC, NSTEP),
        in_specs=[
            pl.BlockSpec((1, BLK2, D), lambda c, i: (c, i, 0)),
            pl.BlockSpec((1, BLK2, D), lambda c, i: (c, i, 0)),
            pl.BlockSpec((1, BLK2, 1), lambda c, i: (c, i, 0)),
            pl.BlockSpec((1, BLK2, G), lambda c, i: (c, i, 0)),
            pl.BlockSpec((NC, G, NS), lambda c, i: (0, 0, 0)),
            pl.BlockSpec((G, 1), lambda c, i: (0, 0)),
            pl.BlockSpec((D, D), lambda c, i: (0, 0)),
            pl.BlockSpec((1, D), lambda c, i: (0, 0)),
            pl.BlockSpec((D, D), lambda c, i: (0, 0)),
            pl.BlockSpec((1, D), lambda c, i: (0, 0)),
            pl.BlockSpec((D, 16), lambda c, i: (0, 0)),
            pl.BlockSpec((1, 16), lambda c, i: (0, 0)),
            pl.BlockSpec((16, C), lambda c, i: (0, 0)),
            pl.BlockSpec((1, C), lambda c, i: (0, 0)),
            pl.BlockSpec((D, 1), lambda c, i: (0, 0)),
            pl.BlockSpec((1, 1), lambda c, i: (0, 0)),
        ],
        out_specs=pl.BlockSpec((1, 3), lambda c, i: (0, 0)),
        out_shape=jax.ShapeDtypeStruct((1, 3), jnp.float32),
        scratch_shapes=[pltpu.VMEM((G, D), jnp.float32),
                        pltpu.VMEM((G, D), jnp.float32)],
    )(agg, xs2, dinvT, QTo, cntT, y2, W1, b1r, W2, b2r,
      Wc1, bc1r, Wc2, bc2r, Wd, bdr)


def kernel(src_x, src_edge_index, src_batch, src_y, tgt_x, tgt_edge_index,
           tgt_batch, W1, b1, W2, b2, Wc1, bc1, Wc2, bc2, Wd, bd):
    srcf = jnp.stack([src_edge_index[0], tgt_edge_index[0]])
    dstf = jnp.stack([src_edge_index[1], tgt_edge_index[1]])
    src1 = srcf.reshape(-1)
    dst1 = dstf.reshape(-1)
    batch1 = jnp.concatenate([src_batch, tgt_batch])
    x2 = jnp.stack([src_x, tgt_x])

    deg_flat, cnt_flat = _sc_deg(dst1, batch1)
    deg_pT = jnp.transpose(deg_flat.reshape(NC, NS, N), (0, 2, 1))
    cntT = jnp.transpose(cnt_flat.reshape(NC, NS, G), (0, 2, 1))

    xs2, dinvT = _tc_scale(deg_pT, x2)
    dinv1 = dinvT.reshape(-1)
    xsf = xs2.reshape(NC * N, D)
    off = jnp.array([[0], [N]], jnp.int32)
    src3 = (srcf + off).reshape(NC, E // RPC, RPC)
    dst3 = dstf.reshape(NC, E // RPC, RPC)
    zrows = jnp.zeros((RPC, D), jnp.float32)

    agg = _sc_agg(xsf, src3, dst3, zrows).reshape(NC, N, D)

    QTo = _sc_qbuild(src1, dst1, batch1, dinv1).reshape(NC, N, G)

    out = _tc_fused(agg, xs2, dinvT, QTo, cntT,
                    src_y.reshape(G, 1).astype(jnp.int32),
                    W1, b1.reshape(1, D), W2, b2.reshape(1, D),
                    Wc1, bc1.reshape(1, 16), Wc2, bc2.reshape(1, C),
                    Wd, bd.reshape(1, 1))
    return (out[0, 0], out[0, 1], out[0, 2])


# speedup vs baseline: 1.0611x; 1.0576x over previous
import functools
import jax
import jax.numpy as jnp
from jax import lax
from jax.experimental import pallas as pl
from jax.experimental.pallas import tpu as pltpu, tpu_sc as plsc

N = 10000
E = 320000
D = 128
G = 128
C = 10

NC = 2
NS = 16
L = 16

ET = E // NS
COLS = N // NS
NB = 624

_mesh = plsc.VectorSubcoreMesh(core_axis_name="c", subcore_axis_name="s")
_sc_params = pltpu.CompilerParams(needs_layout_passes=False)


def _m8(x):
    return pl.multiple_of(x, 8)


@functools.partial(
    pl.kernel,
    out_type=[
        jax.ShapeDtypeStruct((NC * NS * N,), jnp.float32),
        jax.ShapeDtypeStruct((NC * NS * G,), jnp.float32),
    ],
    mesh=_mesh,
    compiler_params=_sc_params,
    scratch_types=[
        pltpu.VMEM((2000,), jnp.int32),
        pltpu.VMEM((N,), jnp.float32),
        pltpu.VMEM((G,), jnp.float32),
    ],
)
def _sc_deg(dst1, batch1, deg_out, cnt_out, stage, deg_loc, cnt_loc):
    c = lax.axis_index("c")
    s = lax.axis_index("s")
    ones = jnp.full((L,), 1.0, jnp.float32)
    zeros = jnp.zeros((L,), jnp.float32)

    def zero_deg(j, _):
        deg_loc[pl.ds(j * L, L)] = zeros
        return 0
    lax.fori_loop(0, N // L, zero_deg, 0)
    for q in range(G // L):
        cnt_loc[pl.ds(q * L, L)] = zeros

    for k in range(ET // 2000):
        pltpu.sync_copy(dst1.at[pl.ds(_m8(c * E + s * ET + k * 2000), 2000)],
                        stage)

        def body(j, _):
            d = stage[pl.ds(j * L, L)]
            plsc.addupdate_scatter(deg_loc, [d], ones)
            return 0
        lax.fori_loop(0, 2000 // L, body, 0)

    pltpu.sync_copy(batch1.at[pl.ds(_m8(c * N + s * NB), NB)],
                    stage.at[pl.ds(0, NB)])

    def bbody(j, _):
        b = stage[pl.ds(j * L, L)]
        plsc.addupdate_scatter(cnt_loc, [b], ones)
        return 0
    lax.fori_loop(0, NB // L, bbody, 0)

    @pl.when(s == NS - 1)
    def _():
        pltpu.sync_copy(batch1.at[pl.ds(_m8(c * N + NS * NB), L)],
                        stage.at[pl.ds(0, L)])
        b = stage[pl.ds(0, L)]
        plsc.addupdate_scatter(cnt_loc, [b], ones)

    pltpu.sync_copy(deg_loc, deg_out.at[pl.ds(_m8((c * NS + s) * N), N)])
    pltpu.sync_copy(cnt_loc, cnt_out.at[pl.ds(_m8((c * NS + s) * G), G)])


RPC = 100
STG = 8
OCH = 7
ORC = RPC
NCH = ET // RPC
NSTG = NCH // STG


@functools.partial(
    pl.kernel,
    out_type=jax.ShapeDtypeStruct((NC, NS * OCH, ORC, D), jnp.float32),
    mesh=_mesh,
    compiler_params=_sc_params,
    scratch_types=[
        pltpu.VMEM((3 * STG, RPC), jnp.int32),
        pltpu.VMEM((3 * STG, RPC), jnp.int32),
        pltpu.VMEM((RPC, D), jnp.float32),
        pltpu.VMEM((RPC, D), jnp.float32),
        pltpu.VMEM((RPC, D), jnp.float32),
        pltpu.VMEM_SHARED((N, D), jnp.float32),
        pltpu.SemaphoreType.DMA,
        pltpu.SemaphoreType.DMA,
        pltpu.SemaphoreType.DMA,
        pltpu.SemaphoreType.DMA,
        pltpu.SemaphoreType.DMA,
        pltpu.SemaphoreType.DMA,
        pltpu.SemaphoreType.DMA,
        pltpu.SemaphoreType.DMA,
        pltpu.SemaphoreType.DMA,
        pltpu.SemaphoreType.DMA,
    ],
)
def _sc_agg(xsf, src3, dst3, zrows, agg_out, st_src, st_dst, rows0, rows1,
            rows2, acc, gs0, gs1, gs2, ss0, ss1, ss2, is0, is1, is2, oc):
    c = lax.axis_index("c")
    s = lax.axis_index("s")

    pltpu.sync_copy(zrows, rows0)
    for i in range(OCH):
        off = min(i * ORC, COLS - ORC)
        pltpu.sync_copy(rows0, acc.at[pl.ds(s * COLS + off, ORC)])
    plsc.subcore_barrier()

    gbufs = (rows0, rows1, rows2)
    gsems = (gs0, gs1, gs2)
    ssems = (ss0, ss1, ss2)
    isems = (is0, is1, is2)

    def stage_idx(kk):
        p = kk % 3
        sb = _m8(s * NCH + kk * STG)
        d0 = pltpu.async_copy(src3.at[c, pl.ds(sb, STG)],
                              st_src.at[pl.ds(p * STG, STG)], isems[p])
        d1 = pltpu.async_copy(dst3.at[c, pl.ds(sb, STG)],
                              st_dst.at[pl.ds(p * STG, STG)], isems[p])
        return (d0, d1)

    def gather(t, b):
        row = (t // STG) % 3 * STG + t % STG
        return pltpu.async_copy(xsf.at[st_src.at[row]], gbufs[b], gsems[b])

    idx_desc = {0: stage_idx(0), 1: stage_idx(1)}
    for dsc in idx_desc[0]:
        dsc.wait()
    gat = [None, None, None]
    sca = [None, None, None]
    gat[0] = gather(0, 0)
    gat[1] = gather(1, 1)
    for t in range(NCH):
        b = t % 3
        g = (t + 2) % 3
        gat[b].wait()
        nt = t + 2
        if nt < NCH and nt % STG == 0:
            kk = nt // STG
            for dsc in idx_desc[kk % 3]:
                dsc.wait()
            if kk + 1 < NSTG:
                idx_desc[(kk + 1) % 3] = stage_idx(kk + 1)
        if sca[g] is not None:
            sca[g].wait()
        if nt < NCH:
            gat[g] = gather(nt, g)
        row = (t // STG) % 3 * STG + t % STG
        sca[b] = pltpu.async_copy(gbufs[b], acc.at[st_dst.at[row]],
                                  ssems[b], add=True)
    sca[(NCH - 1) % 3].wait()

    plsc.subcore_barrier()
    out_desc = [None, None]
    obufs = (rows0, rows1)
    for i in range(OCH):
        off = min(i * ORC, COLS - ORC)
        bb = obufs[i % 2]
        if out_desc[i % 2] is not None:
            out_desc[i % 2].wait()
        pltpu.sync_copy(acc.at[pl.ds(s * COLS + off, ORC)], bb)
        out_desc[i % 2] = pltpu.async_copy(bb, agg_out.at[c, s * OCH + i], oc)
    out_desc[1].wait()
    out_desc[0].wait()


QCH = 2000
NCHQ = ET // QCH
NG = N * G
DUM = 0


@functools.partial(
    pl.kernel,
    out_type=jax.ShapeDtypeStruct((NC * N * G,), jnp.float32),
    mesh=_mesh,
    compiler_params=_sc_params,
    scratch_types=[
        pltpu.VMEM((QCH,), jnp.int32),
        pltpu.VMEM((QCH,), jnp.int32),
        pltpu.VMEM((16, 128), jnp.int32),
        pltpu.VMEM((16, 128), jnp.float32),
        pltpu.VMEM((16000,), jnp.float32),
        pltpu.VMEM((N,), jnp.int32),
        pltpu.VMEM((N,), jnp.float32),
        pltpu.VMEM_SHARED((NG,), jnp.float32),
        pltpu.SemaphoreType.DMA,
        pltpu.SemaphoreType.DMA,
    ],
)
def _sc_qbuild(src1, dst1, batch1, dinv1, q_out, st_s, st_d,
               qi, qv, zb, batch_loc, dinv_loc, qacc, semA, sq):
    c = lax.axis_index("c")
    s = lax.axis_index("s")
    zeros = jnp.zeros((L,), jnp.float32)
    dums = jnp.full((L,), DUM, jnp.int32)

    def zero_zb(j, _):
        zb[pl.ds(j * L, L)] = zeros
        return 0
    lax.fori_loop(0, 16000 // L, zero_zb, 0)
    for i in range(5):
        pltpu.sync_copy(zb, qacc.at[pl.ds(s * 80000 + i * 16000, 16000)])

    pltpu.sync_copy(batch1.at[pl.ds(_m8(c * N), N)], batch_loc)
    pltpu.sync_copy(dinv1.at[pl.ds(_m8(c * N), N)], dinv_loc)
    plsc.subcore_barrier()

    ebase = c * E + s * ET

    def chunk_body(k, _):
        pltpu.sync_copy(src1.at[pl.ds(_m8(ebase + k * QCH), QCH)], st_s)
        pltpu.sync_copy(dst1.at[pl.ds(_m8(ebase + k * QCH), QCH)], st_d)

        @pl.when(k > 0)
        def _():
            for r in range(16):
                pltpu.make_async_copy(qv.at[r], qacc.at[qi.at[r]], sq).wait()

        for j in range(QCH // L):
            r, col = j >> 3, (j & 7) * L
            sv = st_s[pl.ds(j * L, L)]
            dv = st_d[pl.ds(j * L, L)]
            g = plsc.load_gather(batch_loc, [dv])
            f2 = plsc.load_gather(dinv_loc, [dv])
            qi[r, pl.ds(col, L)] = lax.shift_left(sv, 7) + g
            qv[r, pl.ds(col, L)] = f2
        for j in range(QCH // L, 128):
            r, col = j >> 3, (j & 7) * L
            qi[r, pl.ds(col, L)] = dums
            qv[r, pl.ds(col, L)] = zeros

        for r in range(16):
            pltpu.async_copy(qv.at[r], qacc.at[qi.at[r]], sq, add=True)
        return 0
    lax.fori_loop(0, NCHQ, chunk_body, 0)
    for r in range(16):
        pltpu.make_async_copy(qv.at[r], qacc.at[qi.at[r]], sq).wait()

    io = lax.iota(jnp.int32, L)
    base = s * COLS
    for j in range(40):
        r, col = j >> 3, (j & 7) * L
        jv = base + j * L + io
        ok = (jv - base) < COLS
        jvc = jnp.where(jv > N - 1, N - 1, jv)
        g = plsc.load_gather(batch_loc, [jvc])
        dvv = plsc.load_gather(dinv_loc, [jvc])
        qi[r, pl.ds(col, L)] = jnp.where(ok, lax.shift_left(jv, 7) + g, DUM)
        qv[r, pl.ds(col, L)] = jnp.where(ok, dvv, 0.0)
    for r in range(5):
        pltpu.async_copy(qv.at[r], qacc.at[qi.at[r]], sq, add=True)
    for r in range(5):
        pltpu.make_async_copy(qv.at[r], qacc.at[qi.at[r]], sq).wait()

    plsc.subcore_barrier()
    for i in range(5):
        pltpu.sync_copy(qacc.at[pl.ds(s * 80000 + i * 16000, 16000)], zb)
        pltpu.sync_copy(zb, q_out.at[pl.ds(_m8(c * NG + s * 80000 + i * 16000),
                                           16000)])


BLK1 = 2000


def _tc1_body(degp_ref, x_ref, xs_ref, dinv_ref):
    deg = jnp.sum(degp_ref[0], axis=1, keepdims=True) + 1.0
    dinv = lax.rsqrt(jnp.maximum(deg, 1.0))
    xs_ref[0] = x_ref[0] * dinv
    dinv_ref[0] = dinv


def _tc_scale(deg_pT, x2):
    return pl.pallas_call(
        _tc1_body,
        grid=(NC, N // BLK1),
        in_specs=[
            pl.BlockSpec((1, BLK1, NS), lambda c, i: (c, i, 0)),
            pl.BlockSpec((1, BLK1, D), lambda c, i: (c, i, 0)),
        ],
        out_specs=[
            pl.BlockSpec((1, BLK1, D), lambda c, i: (c, i, 0)),
            pl.BlockSpec((1, BLK1, 1), lambda c, i: (c, i, 0)),
        ],
        out_shape=[
            jax.ShapeDtypeStruct((NC, N, D), jnp.float32),
            jax.ShapeDtypeStruct((NC, N, 1), jnp.float32),
        ],
    )(deg_pT, x2)


BLK2 = 2000
NSTEP = N // BLK2
EPS = 1e-7


def _bce_mean(p, y):
    p = jnp.clip(p, EPS, 1.0 - EPS)
    return -jnp.mean(y * jnp.log(p) + (1.0 - y) * jnp.log(1.0 - p))


def _tc23_body(agg_ref, xs_ref, dinv_ref, q_ref, cntT_ref, y_ref,
               w1_ref, b1_ref, w2_ref, b2_ref, wc1_ref, bc1_ref,
               wc2_ref, bc2_ref, wd_ref, bd_ref, out_ref, acc0, acc1):
    c = pl.program_id(0)
    i = pl.program_id(1)

    @pl.when((c == 0) & (i == 0))
    def _():
        acc0[...] = jnp.zeros((G, D), jnp.float32)
        acc1[...] = jnp.zeros((G, D), jnp.float32)

    a = (agg_ref[0] + xs_ref[0]) * dinv_ref[0]
    h = lax.dot_general(a, w1_ref[...], (((1,), (0,)), ((), ())),
                        preferred_element_type=jnp.float32) + b1_ref[...]
    h = jnp.maximum(h, 0.0)
    m = lax.dot_general(h, w2_ref[...], (((1,), (0,)), ((), ())),
                        preferred_element_type=jnp.float32)
    m = m * dinv_ref[0]
    qm = lax.dot_general(q_ref[0], m, (((0,), (0,)), ((), ())),
                         preferred_element_type=jnp.float32)

    @pl.when(c == 0)
    def _():
        acc0[...] += qm

    @pl.when(c == 1)
    def _():
        acc1[...] += qm

    @pl.when((c == NC - 1) & (i == NSTEP - 1))
    def _():
        cnt_s = jnp.maximum(jnp.sum(cntT_ref[0], axis=1, keepdims=True), 1.0)
        cnt_t = jnp.maximum(jnp.sum(cntT_ref[1], axis=1, keepdims=True), 1.0)
        ps = acc0[...] / cnt_s + b2_ref[...]
        pt = acc1[...] / cnt_t + b2_ref[...]

        hh = lax.dot_general(ps, wc1_ref[...], (((1,), (0,)), ((), ())),
                             preferred_element_type=jnp.float32) + bc1_ref[...]
        hh = jnp.maximum(hh, 0.0)
        z = lax.dot_general(hh, wc2_ref[...], (((1,), (0,)), ((), ())),
                            preferred_element_type=jnp.float32) + bc2_ref[...]
        logits = jax.nn.sigmoid(z)
        ycol = y_ref[...]
        onehot = (lax.broadcasted_iota(jnp.int32, (G, C), 1) == ycol
                  ).astype(jnp.float32)
        clf = _bce_mean(logits, onehot)

        sp = jax.nn.sigmoid(
            lax.dot_general(ps, wd_ref[...], (((1,), (0,)), ((), ())),
                            preferred_element_type=jnp.float32) + bd_ref[...])
        tp = jax.nn.sigmoid(
            lax.dot_general(pt, wd_ref[...], (((1,), (0,)), ((), ())),
                            preferred_element_type=jnp.float32) + bd_ref[...])
        dl = _bce_mean(sp, jnp.zeros_like(sp)) + _bce_mean(tp, jnp.ones_like(tp))
        total = clf + dl
        out_ref[...] = jnp.stack([total, clf, dl]).reshape(1, 3)


def _tc_fused(agg, xs2, dinvT, QTo, cntT, y2, W1, b1r, W2, b2r,
              Wc1, bc1r, Wc2, bc2r, Wd, bdr):
    return pl.pallas_call(
        _tc23_body,
        grid=(NC, NSTEP),
        in_specs=[
            pl.BlockSpec((1, BLK2, D), lambda c, i: (c, i, 0)),
            pl.BlockSpec((1, BLK2, D), lambda c, i: (c, i, 0)),
            pl.BlockSpec((1, BLK2, 1), lambda c, i: (c, i, 0)),
            pl.BlockSpec((1, BLK2, G), lambda c, i: (c, i, 0)),
            pl.BlockSpec((NC, G, NS), lambda c, i: (0, 0, 0)),
            pl.BlockSpec((G, 1), lambda c, i: (0, 0)),
            pl.BlockSpec((D, D), lambda c, i: (0, 0)),
            pl.BlockSpec((1, D), lambda c, i: (0, 0)),
            pl.BlockSpec((D, D), lambda c, i: (0, 0)),
            pl.BlockSpec((1, D), lambda c, i: (0, 0)),
            pl.BlockSpec((D, 16), lambda c, i: (0, 0)),
            pl.BlockSpec((1, 16), lambda c, i: (0, 0)),
            pl.BlockSpec((16, C), lambda c, i: (0, 0)),
            pl.BlockSpec((1, C), lambda c, i: (0, 0)),
            pl.BlockSpec((D, 1), lambda c, i: (0, 0)),
            pl.BlockSpec((1, 1), lambda c, i: (0, 0)),
        ],
        out_specs=pl.BlockSpec((1, 3), lambda c, i: (0, 0)),
        out_shape=jax.ShapeDtypeStruct((1, 3), jnp.float32),
        scratch_shapes=[pltpu.VMEM((G, D), jnp.float32),
                        pltpu.VMEM((G, D), jnp.float32)],
    )(agg, xs2, dinvT, QTo, cntT, y2, W1, b1r, W2, b2r,
      Wc1, bc1r, Wc2, bc2r, Wd, bdr)


def kernel(src_x, src_edge_index, src_batch, src_y, tgt_x, tgt_edge_index,
           tgt_batch, W1, b1, W2, b2, Wc1, bc1, Wc2, bc2, Wd, bd):
    srcf = jnp.stack([src_edge_index[0], tgt_edge_index[0]])
    dstf = jnp.stack([src_edge_index[1], tgt_edge_index[1]])
    src1 = srcf.reshape(-1)
    dst1 = dstf.reshape(-1)
    batch1 = jnp.concatenate([src_batch, tgt_batch])
    x2 = jnp.stack([src_x, tgt_x])

    deg_flat, cnt_flat = _sc_deg(dst1, batch1)
    deg_pT = jnp.transpose(deg_flat.reshape(NC, NS, N), (0, 2, 1))
    cntT = jnp.transpose(cnt_flat.reshape(NC, NS, G), (0, 2, 1))

    xs2, dinvT = _tc_scale(deg_pT, x2)
    dinv1 = dinvT.reshape(-1)
    xsf = xs2.reshape(NC * N, D)
    off = jnp.array([[0], [N]], jnp.int32)
    src3 = (srcf + off).reshape(NC, E // RPC, RPC)
    dst3 = dstf.reshape(NC, E // RPC, RPC)
    zrows = jnp.zeros((RPC, D), jnp.float32)

    agg7 = _sc_agg(xsf, src3, dst3, zrows).reshape(NC, NS, OCH, ORC, D)
    agg = jnp.concatenate(
        [agg7[:, :, :6].reshape(NC, NS, 600, D), agg7[:, :, 6, 75:100]],
        axis=2).reshape(NC, N, D)

    QTo = _sc_qbuild(src1, dst1, batch1, dinv1).reshape(NC, N, G)

    out = _tc_fused(agg, xs2, dinvT, QTo, cntT,
                    src_y.reshape(G, 1).astype(jnp.int32),
                    W1, b1.reshape(1, D), W2, b2.reshape(1, D),
                    Wc1, bc1.reshape(1, 16), Wc2, bc2.reshape(1, C),
                    Wd, bd.reshape(1, 1))
    return (out[0, 0], out[0, 1], out[0, 2])
